# Initial kernel scaffold; baseline (speedup 1.0000x reference)
import jax
import jax.numpy as jnp
from jax.experimental import pallas as pl


def kernel(hidden_states, in_proj_w, conv_w, A_log, dt_bias, norm_w, out_proj_w):
    raise NotImplementedError("write your pallas kernel here")


# speedup vs baseline: 6.0264x; 6.0264x over previous
import jax
import jax.numpy as jnp
from jax.experimental import pallas as pl
from jax.experimental.pallas import tpu as pltpu

H = 2048
HK, HV, G = 16, 32, 2
DK = DV = 128
KEY_DIM = HK * DK
VAL_DIM = HV * DV
QKVZ = 2 * KEY_DIM + 2 * VAL_DIM
NPROJ = QKVZ + 2 * HV
NPAD = 12544
L = 4096
KCONV = 4
EPS = 1e-6
C = 64
NC = L // C
PER_HEAD = 2 * DK + G * DV + G * DV


def _matmul_body(x_ref, w_ref, o_ref):
    o_ref[...] = jnp.dot(x_ref[...], w_ref[...],
                         preferred_element_type=jnp.float32)


def _in_proj(hidden, w_pad):
    return pl.pallas_call(
        _matmul_body,
        grid=(14, 8),
        in_specs=[pl.BlockSpec((512, H), lambda n, m: (m, 0)),
                  pl.BlockSpec((H, 896), lambda n, m: (0, n))],
        out_specs=pl.BlockSpec((512, 896), lambda n, m: (m, n)),
        out_shape=jax.ShapeDtypeStruct((L, NPAD), jnp.float32),
        compiler_params=pltpu.CompilerParams(
            dimension_semantics=("parallel", "arbitrary")),
        name="gdn_in_proj",
    )(hidden, w_pad)


def _out_proj(core, w):
    return pl.pallas_call(
        _matmul_body,
        grid=(4, 8),
        in_specs=[pl.BlockSpec((512, VAL_DIM), lambda n, m: (m, 0)),
                  pl.BlockSpec((VAL_DIM, 512), lambda n, m: (0, n))],
        out_specs=pl.BlockSpec((512, 512), lambda n, m: (m, n)),
        out_shape=jax.ShapeDtypeStruct((L, H), jnp.float32),
        compiler_params=pltpu.CompilerParams(
            dimension_semantics=("parallel", "arbitrary")),
        name="gdn_out_proj",
    )(core, w)


def _gdn_body(qkvz_ref, ba_ref, convw_ref, gate_ref, normw_ref, out_ref,
              s_ref, carry_ref):
    kh = pl.program_id(0)
    s_ref[...] = jnp.zeros_like(s_ref)
    carry_ref[...] = jnp.zeros_like(carry_ref)

    ii = jax.lax.broadcasted_iota(jnp.int32, (C, C), 0)
    jj = jax.lax.broadcasted_iota(jnp.int32, (C, C), 1)
    tril_incl = (ii >= jj).astype(jnp.float32)
    strict = ii > jj
    incl = ii >= jj
    eye = (ii == jj).astype(jnp.float32)
    lane = jax.lax.broadcasted_iota(jnp.int32, (1, 128), 1)
    cw = convw_ref[0]
    nw = normw_ref[...]
    hi = jax.lax.Precision.HIGHEST

    def body(i, _):
        blk = qkvz_ref[pl.ds(i * C, C), :]
        x = blk[:, :512]
        z = blk[:, 512:768]
        prev = carry_ref[...]
        carry_ref[...] = x[56:64, :]
        xe = jnp.concatenate([prev[5:8, :], x], axis=0)
        co = (cw[0:1, :] * xe[0:64, :] + cw[1:2, :] * xe[1:65, :]
              + cw[2:3, :] * xe[2:66, :] + cw[3:4, :] * xe[3:67, :])
        co = co * jax.nn.sigmoid(co)
        mq = co[:, 0:128]
        mk = co[:, 128:256]
        qn = mq * jax.lax.rsqrt(
            jnp.sum(mq * mq, axis=1, keepdims=True) + EPS) * (DK ** -0.5)
        kn = mk * jax.lax.rsqrt(
            jnp.sum(mk * mk, axis=1, keepdims=True) + EPS)
        bablk = ba_ref[pl.ds(i * C, C), :]

        for vh in range(G):
            h = G * kh + vh
            bcol = jnp.sum(jnp.where(lane == 4 * kh + vh, bablk, 0.0),
                           axis=1, keepdims=True)
            acol = jnp.sum(jnp.where(lane == 4 * kh + 2 + vh, bablk, 0.0),
                           axis=1, keepdims=True)
            beta = jax.nn.sigmoid(bcol)
            g = gate_ref[0, h] * jax.nn.softplus(acol + gate_ref[1, h])
            c = jnp.dot(tril_incl, g, precision=hi,
                        preferred_element_type=jnp.float32)
            c_row = jax.lax.dot_general(
                g, tril_incl, (((0,), (1,)), ((), ())),
                precision=hi, preferred_element_type=jnp.float32)
            D = jnp.exp(c - c_row)
            ec = jnp.exp(c)
            kb = kn * beta
            ap = jax.lax.dot_general(
                kb, kn, (((1,), (1,)), ((), ())),
                preferred_element_type=jnp.float32)
            B = jnp.where(strict, -ap * D, 0.0)
            T = eye + B
            P = jnp.dot(B, B, preferred_element_type=jnp.float32)
            for it in range(5):
                T = T + jnp.dot(T, P, preferred_element_type=jnp.float32)
                if it < 4:
                    P = jnp.dot(P, P, preferred_element_type=jnp.float32)
            vv = co[:, 256 + 128 * vh:384 + 128 * vh]
            u = jnp.dot(T, beta * vv, preferred_element_type=jnp.float32)
            w = jnp.dot(T, kb * ec, preferred_element_type=jnp.float32)
            S = s_ref[vh]
            U = u - jnp.dot(w, S, preferred_element_type=jnp.float32)
            attn = jnp.where(
                incl,
                jax.lax.dot_general(qn, kn, (((1,), (1,)), ((), ())),
                                    preferred_element_type=jnp.float32) * D,
                0.0)
            o = (jnp.dot(qn * ec, S, preferred_element_type=jnp.float32)
                 + jnp.dot(attn, U, preferred_element_type=jnp.float32))
            ecl = jnp.exp(c[63:64, :])
            dk = jnp.exp(c[63:64, :] - c)
            s_ref[vh] = ecl * S + jax.lax.dot_general(
                kn * dk, U, (((0,), (0,)), ((), ())),
                preferred_element_type=jnp.float32)
            var = jnp.mean(o * o, axis=1, keepdims=True)
            on = o * jax.lax.rsqrt(var + EPS) * nw
            zz = z[:, 128 * vh:128 * (vh + 1)]
            out_ref[pl.ds(i * C, C), 128 * vh:128 * (vh + 1)] = (
                on * (zz * jax.nn.sigmoid(zz)))
        return 0

    jax.lax.fori_loop(0, NC, body, 0)


def _gdn_core(proj, convw_r, gate, normw2):
    return pl.pallas_call(
        _gdn_body,
        grid=(HK,),
        in_specs=[
            pl.BlockSpec((L, PER_HEAD), lambda h: (0, h)),
            pl.BlockSpec((L, 128), lambda h: (0, 96)),
            pl.BlockSpec((1, 4, 512), lambda h: (h, 0, 0)),
            pl.BlockSpec(memory_space=pltpu.SMEM),
            pl.BlockSpec((1, 128), lambda h: (0, 0)),
        ],
        out_specs=pl.BlockSpec((L, G * DV), lambda h: (0, h)),
        out_shape=jax.ShapeDtypeStruct((L, VAL_DIM), jnp.float32),
        scratch_shapes=[
            pltpu.VMEM((G, DK, DV), jnp.float32),
            pltpu.VMEM((8, 512), jnp.float32),
        ],
        compiler_params=pltpu.CompilerParams(
            dimension_semantics=("parallel",),
            vmem_limit_bytes=50 * 1024 * 1024),
        name="gdn_core",
    )(proj, proj, convw_r, gate, normw2)


def kernel(hidden_states, in_proj_w, conv_w, A_log, dt_bias, norm_w,
           out_proj_w):
    w_pad = jnp.pad(in_proj_w, ((0, 0), (0, NPAD - NPROJ)))
    proj = _in_proj(hidden_states, w_pad)

    cwt = conv_w.T
    cq = cwt[:, :KEY_DIM].reshape(4, HK, DK).transpose(1, 0, 2)
    ck = cwt[:, KEY_DIM:2 * KEY_DIM].reshape(4, HK, DK).transpose(1, 0, 2)
    cv = cwt[:, 2 * KEY_DIM:].reshape(4, HK, G * DV).transpose(1, 0, 2)
    convw_r = jnp.concatenate([cq, ck, cv], axis=2)

    gate = jnp.stack([-jnp.exp(A_log), dt_bias])
    normw2 = norm_w.reshape(1, DV)

    core = _gdn_core(proj, convw_r, gate, normw2)
    return _out_proj(core, out_proj_w)
